# baseline (speedup 1.0000x reference)
import functools

import jax
import jax.numpy as jnp
from jax import lax
from jax.experimental import pallas as pl
from jax.experimental.pallas import tpu as pltpu
from jax.experimental.pallas import tpu_sc as plsc

_ALPHA = 0.5


def _make_gather(C, D, B):
    NC, NS = 2, 16
    NW = NC * NS
    b_per_w = B // NW
    mesh = plsc.VectorSubcoreMesh(core_axis_name="c", subcore_axis_name="s")

    @functools.partial(
        pl.kernel,
        out_type=jax.ShapeDtypeStruct((B, D), jnp.float32),
        mesh=mesh,
        scratch_types=[
            pltpu.VMEM((b_per_w,), jnp.int32),
            pltpu.VMEM((b_per_w, D), jnp.float32),
            pltpu.SemaphoreType.DMA,
        ],
    )
    def gather_k(centers_hbm, idx_hbm, out_hbm, idx_v, rows_v, sem):
        wid = lax.axis_index("s") * NC + lax.axis_index("c")
        base = wid * b_per_w
        pltpu.sync_copy(idx_hbm.at[pl.ds(base, b_per_w)], idx_v)
        pltpu.async_copy(centers_hbm.at[idx_v], rows_v, sem).wait()
        pltpu.sync_copy(rows_v, out_hbm.at[pl.ds(base, b_per_w)])

    return gather_k


_BI = 512


def _tc_body(lcol_ref, lrow_ref, f_ref, cb_ref, res_ref, u_ref):
    B, D = f_ref.shape
    lrow = lrow_ref[...]
    d_all = cb_ref[...] - f_ref[...]
    for i in range(B // _BI):
        sl = pl.ds(i * _BI, _BI)
        lcol = lcol_ref[sl, :]
        eqf = (lcol == lrow).astype(jnp.float32)
        appear = jnp.sum(eqf, axis=1, keepdims=True)
        m = jax.lax.dot_general(
            eqf, d_all, (((1,), (0,)), ((), ())),
            preferred_element_type=jnp.float32)
        scale = _ALPHA / (1.0 + appear)
        cb_blk = cb_ref[sl, :]
        u_ref[sl, :] = cb_blk - scale * m
        r = f_ref[sl, :] - cb_blk
        res_ref[sl, :] = jnp.sum(r * r, axis=1, keepdims=True)


def _tc_math(labels, features, cb):
    B, D = features.shape
    lcol = labels.reshape(B, 1)
    lrow = labels.reshape(1, B)
    return pl.pallas_call(
        _tc_body,
        in_specs=[
            pl.BlockSpec((B, 1), lambda: (0, 0)),
            pl.BlockSpec((1, B), lambda: (0, 0)),
            pl.BlockSpec((B, D), lambda: (0, 0)),
            pl.BlockSpec((B, D), lambda: (0, 0)),
        ],
        out_specs=[
            pl.BlockSpec((B, 1), lambda: (0, 0)),
            pl.BlockSpec((B, D), lambda: (0, 0)),
        ],
        out_shape=[
            jax.ShapeDtypeStruct((B, 1), jnp.float32),
            jax.ShapeDtypeStruct((B, D), jnp.float32),
        ],
    )(lcol, lrow, features, cb)


_NDMA = 8


def _copy_body(src_ref, dst_ref, *sems):
    C = src_ref.shape[0]
    R = C // _NDMA
    cps = [
        pltpu.make_async_copy(
            src_ref.at[pl.ds(i * R, R)], dst_ref.at[pl.ds(i * R, R)], sems[i])
        for i in range(_NDMA)
    ]
    for cp in cps:
        cp.start()
    for cp in cps:
        cp.wait()


def _tc_copy(centers):
    C, D = centers.shape
    return pl.pallas_call(
        _copy_body,
        in_specs=[pl.BlockSpec(memory_space=pl.ANY)],
        out_specs=pl.BlockSpec(memory_space=pl.ANY),
        out_shape=jax.ShapeDtypeStruct((C, D), jnp.float32),
        scratch_shapes=[pltpu.SemaphoreType.DMA] * _NDMA,
    )(centers)


def _make_scatter(C, D, B):
    NC, NS = 2, 16
    NW = NC * NS
    b_per_w = B // NW
    mesh = plsc.VectorSubcoreMesh(core_axis_name="c", subcore_axis_name="s")

    @functools.partial(
        pl.kernel,
        out_type=jax.ShapeDtypeStruct((b_per_w,), jnp.int32),
        mesh=mesh,
        scratch_types=[
            pltpu.VMEM((b_per_w,), jnp.int32),
            pltpu.VMEM((b_per_w, D), jnp.float32),
            pltpu.SemaphoreType.DMA,
        ],
        compiler_params=pltpu.CompilerParams(has_side_effects=True),
    )
    def scatter_k(table_hbm, idx_hbm, val_hbm, tok_hbm, idx_v, rows_v, sem):
        cid = lax.axis_index("c")
        sid = lax.axis_index("s")
        wid = sid * NC + cid
        base = wid * b_per_w
        pltpu.sync_copy(idx_hbm.at[pl.ds(base, b_per_w)], idx_v)
        pltpu.sync_copy(val_hbm.at[pl.ds(base, b_per_w)], rows_v)
        pltpu.async_copy(rows_v, table_hbm.at[idx_v], sem).wait()

        @pl.when(wid == 0)
        def _():
            pltpu.sync_copy(idx_v, tok_hbm)

    return scatter_k


def kernel(features, labels, centers):
    labels = labels.reshape(-1).astype(jnp.int32)
    features = features.astype(jnp.float32)
    B, D = features.shape
    C = centers.shape[0]

    cb = _make_gather(C, D, B)(centers, labels)
    result, u = _tc_math(labels, features, cb)
    table = _tc_copy(centers)
    tok = _make_scatter(C, D, B)(table, labels, u)
    table, _ = lax.optimization_barrier((table, tok))
    return (result, table)


# speedup vs baseline: 19.8509x; 19.8509x over previous
import functools

import jax
import jax.numpy as jnp
from jax import lax
from jax.experimental import pallas as pl
from jax.experimental.pallas import tpu as pltpu
from jax.experimental.pallas import tpu_sc as plsc

_ALPHA = 0.5


def _make_gather(C, D, B):
    NC, NS = 2, 16
    NW = NC * NS
    b_per_w = B // NW
    mesh = plsc.VectorSubcoreMesh(core_axis_name="c", subcore_axis_name="s")

    @functools.partial(
        pl.kernel,
        out_type=jax.ShapeDtypeStruct((B, D), jnp.float32),
        mesh=mesh,
        scratch_types=[
            pltpu.VMEM((b_per_w,), jnp.int32),
            pltpu.VMEM((b_per_w, D), jnp.float32),
            pltpu.SemaphoreType.DMA,
        ],
    )
    def gather_k(centers_hbm, idx_hbm, out_hbm, idx_v, rows_v, sem):
        wid = lax.axis_index("s") * NC + lax.axis_index("c")
        base = wid * b_per_w
        pltpu.sync_copy(idx_hbm.at[pl.ds(base, b_per_w)], idx_v)
        pltpu.async_copy(centers_hbm.at[idx_v], rows_v, sem).wait()
        pltpu.sync_copy(rows_v, out_hbm.at[pl.ds(base, b_per_w)])

    return gather_k


_BI = 512


def _tc_body(lcol_ref, lrow_ref, f_ref, cb_ref, res_ref, u_ref):
    B, D = f_ref.shape
    lrow = lrow_ref[...]
    d_all = cb_ref[...] - f_ref[...]
    for i in range(B // _BI):
        sl = pl.ds(i * _BI, _BI)
        lcol = lcol_ref[sl, :]
        eqf = (lcol == lrow).astype(jnp.float32)
        appear = jnp.sum(eqf, axis=1, keepdims=True)
        m = jax.lax.dot_general(
            eqf, d_all, (((1,), (0,)), ((), ())),
            preferred_element_type=jnp.float32)
        scale = _ALPHA / (1.0 + appear)
        cb_blk = cb_ref[sl, :]
        u_ref[sl, :] = cb_blk - scale * m
        r = f_ref[sl, :] - cb_blk
        res_ref[sl, :] = jnp.sum(r * r, axis=1, keepdims=True)


def _tc_math(labels, features, cb):
    B, D = features.shape
    lcol = labels.reshape(B, 1)
    lrow = labels.reshape(1, B)
    return pl.pallas_call(
        _tc_body,
        in_specs=[
            pl.BlockSpec((B, 1), lambda: (0, 0)),
            pl.BlockSpec((1, B), lambda: (0, 0)),
            pl.BlockSpec((B, D), lambda: (0, 0)),
            pl.BlockSpec((B, D), lambda: (0, 0)),
        ],
        out_specs=[
            pl.BlockSpec((B, 1), lambda: (0, 0)),
            pl.BlockSpec((B, D), lambda: (0, 0)),
        ],
        out_shape=[
            jax.ShapeDtypeStruct((B, 1), jnp.float32),
            jax.ShapeDtypeStruct((B, D), jnp.float32),
        ],
    )(lcol, lrow, features, cb)


_BR = 4000


def _copy_body(src_ref, dst_ref):
    dst_ref[...] = src_ref[...]


def _tc_copy(centers):
    C, D = centers.shape
    return pl.pallas_call(
        _copy_body,
        grid=(C // _BR,),
        in_specs=[pl.BlockSpec((_BR, D), lambda i: (i, 0))],
        out_specs=pl.BlockSpec((_BR, D), lambda i: (i, 0)),
        out_shape=jax.ShapeDtypeStruct((C, D), jnp.float32),
    )(centers)


def _make_scatter(C, D, B):
    NC, NS = 2, 16
    NW = NC * NS
    b_per_w = B // NW
    mesh = plsc.VectorSubcoreMesh(core_axis_name="c", subcore_axis_name="s")

    @functools.partial(
        pl.kernel,
        out_type=jax.ShapeDtypeStruct((b_per_w,), jnp.int32),
        mesh=mesh,
        scratch_types=[
            pltpu.VMEM((b_per_w,), jnp.int32),
            pltpu.VMEM((b_per_w, D), jnp.float32),
            pltpu.SemaphoreType.DMA,
        ],
        compiler_params=pltpu.CompilerParams(has_side_effects=True),
    )
    def scatter_k(table_hbm, idx_hbm, val_hbm, tok_hbm, idx_v, rows_v, sem):
        cid = lax.axis_index("c")
        sid = lax.axis_index("s")
        wid = sid * NC + cid
        base = wid * b_per_w
        pltpu.sync_copy(idx_hbm.at[pl.ds(base, b_per_w)], idx_v)
        pltpu.sync_copy(val_hbm.at[pl.ds(base, b_per_w)], rows_v)
        pltpu.async_copy(rows_v, table_hbm.at[idx_v], sem).wait()

        @pl.when(wid == 0)
        def _():
            pltpu.sync_copy(idx_v, tok_hbm)

    return scatter_k


def kernel(features, labels, centers):
    labels = labels.reshape(-1).astype(jnp.int32)
    features = features.astype(jnp.float32)
    B, D = features.shape
    C = centers.shape[0]

    cb = _make_gather(C, D, B)(centers, labels)
    result, u = _tc_math(labels, features, cb)
    table = _tc_copy(centers)
    tok = _make_scatter(C, D, B)(table, labels, u)
    table, _ = lax.optimization_barrier((table, tok))
    return (result, table)


# speedup vs baseline: 20.7958x; 1.0476x over previous
import functools

import jax
import jax.numpy as jnp
from jax import lax
from jax.experimental import pallas as pl
from jax.experimental.pallas import tpu as pltpu
from jax.experimental.pallas import tpu_sc as plsc

_ALPHA = 0.5


def _make_gather(C, D, B):
    NC, NS = 2, 16
    NW = NC * NS
    b_per_w = B // NW
    mesh = plsc.VectorSubcoreMesh(core_axis_name="c", subcore_axis_name="s")

    @functools.partial(
        pl.kernel,
        out_type=jax.ShapeDtypeStruct((B, D), jnp.float32),
        mesh=mesh,
        scratch_types=[
            pltpu.VMEM((b_per_w,), jnp.int32),
            pltpu.VMEM((b_per_w, D), jnp.float32),
            pltpu.SemaphoreType.DMA,
        ],
    )
    def gather_k(centers_hbm, idx_hbm, out_hbm, idx_v, rows_v, sem):
        wid = lax.axis_index("s") * NC + lax.axis_index("c")
        base = wid * b_per_w
        pltpu.sync_copy(idx_hbm.at[pl.ds(base, b_per_w)], idx_v)
        pltpu.async_copy(centers_hbm.at[idx_v], rows_v, sem).wait()
        pltpu.sync_copy(rows_v, out_hbm.at[pl.ds(base, b_per_w)])

    return gather_k


_BI = 512


def _tc_body(lcol_ref, lrow_ref, f_ref, cb_ref, res_ref, u_ref):
    B, D = f_ref.shape
    lrow = lrow_ref[...]
    d_all = cb_ref[...] - f_ref[...]
    for i in range(B // _BI):
        sl = pl.ds(i * _BI, _BI)
        lcol = lcol_ref[sl, :]
        eqf = (lcol == lrow).astype(jnp.float32)
        appear = jnp.sum(eqf, axis=1, keepdims=True)
        m = jax.lax.dot_general(
            eqf, d_all, (((1,), (0,)), ((), ())),
            preferred_element_type=jnp.float32)
        scale = _ALPHA / (1.0 + appear)
        cb_blk = cb_ref[sl, :]
        u_ref[sl, :] = cb_blk - scale * m
        r = f_ref[sl, :] - cb_blk
        res_ref[sl, :] = jnp.sum(r * r, axis=1, keepdims=True)


def _tc_math(labels, features, cb):
    B, D = features.shape
    lcol = labels.reshape(B, 1)
    lrow = labels.reshape(1, B)
    return pl.pallas_call(
        _tc_body,
        in_specs=[
            pl.BlockSpec((B, 1), lambda: (0, 0)),
            pl.BlockSpec((1, B), lambda: (0, 0)),
            pl.BlockSpec((B, D), lambda: (0, 0)),
            pl.BlockSpec((B, D), lambda: (0, 0)),
        ],
        out_specs=[
            pl.BlockSpec((B, 1), lambda: (0, 0)),
            pl.BlockSpec((B, D), lambda: (0, 0)),
        ],
        out_shape=[
            jax.ShapeDtypeStruct((B, 1), jnp.float32),
            jax.ShapeDtypeStruct((B, D), jnp.float32),
        ],
    )(lcol, lrow, features, cb)


_BR = 10000


def _copy_body(src_ref, dst_ref):
    dst_ref[...] = src_ref[...]


def _tc_copy(centers):
    C, D = centers.shape
    return pl.pallas_call(
        _copy_body,
        grid=(C // _BR,),
        in_specs=[pl.BlockSpec((_BR, D), lambda i: (i, 0))],
        out_specs=pl.BlockSpec((_BR, D), lambda i: (i, 0)),
        out_shape=jax.ShapeDtypeStruct((C, D), jnp.float32),
    )(centers)


def _make_scatter(C, D, B):
    NC, NS = 2, 16
    NW = NC * NS
    b_per_w = B // NW
    mesh = plsc.VectorSubcoreMesh(core_axis_name="c", subcore_axis_name="s")

    @functools.partial(
        pl.kernel,
        out_type=jax.ShapeDtypeStruct((b_per_w,), jnp.int32),
        mesh=mesh,
        scratch_types=[
            pltpu.VMEM((b_per_w,), jnp.int32),
            pltpu.VMEM((b_per_w, D), jnp.float32),
            pltpu.SemaphoreType.DMA,
        ],
        compiler_params=pltpu.CompilerParams(has_side_effects=True),
    )
    def scatter_k(table_hbm, idx_hbm, val_hbm, tok_hbm, idx_v, rows_v, sem):
        cid = lax.axis_index("c")
        sid = lax.axis_index("s")
        wid = sid * NC + cid
        base = wid * b_per_w
        pltpu.sync_copy(idx_hbm.at[pl.ds(base, b_per_w)], idx_v)
        pltpu.sync_copy(val_hbm.at[pl.ds(base, b_per_w)], rows_v)
        pltpu.async_copy(rows_v, table_hbm.at[idx_v], sem).wait()

        @pl.when(wid == 0)
        def _():
            pltpu.sync_copy(idx_v, tok_hbm)

    return scatter_k


def kernel(features, labels, centers):
    labels = labels.reshape(-1).astype(jnp.int32)
    features = features.astype(jnp.float32)
    B, D = features.shape
    C = centers.shape[0]

    cb = _make_gather(C, D, B)(centers, labels)
    result, u = _tc_math(labels, features, cb)
    table = _tc_copy(centers)
    tok = _make_scatter(C, D, B)(table, labels, u)
    table, _ = lax.optimization_barrier((table, tok))
    return (result, table)


# speedup vs baseline: 21.2742x; 1.0230x over previous
import functools

import jax
import jax.numpy as jnp
from jax import lax
from jax.experimental import pallas as pl
from jax.experimental.pallas import tpu as pltpu
from jax.experimental.pallas import tpu_sc as plsc

_ALPHA = 0.5


def _make_gather(C, D, B):
    NC, NS = 2, 16
    NW = NC * NS
    b_per_w = B // NW
    mesh = plsc.VectorSubcoreMesh(core_axis_name="c", subcore_axis_name="s")

    @functools.partial(
        pl.kernel,
        out_type=jax.ShapeDtypeStruct((B, D), jnp.float32),
        mesh=mesh,
        scratch_types=[
            pltpu.VMEM((b_per_w,), jnp.int32),
            pltpu.VMEM((b_per_w, D), jnp.float32),
            pltpu.SemaphoreType.DMA,
        ],
    )
    def gather_k(centers_hbm, idx_hbm, out_hbm, idx_v, rows_v, sem):
        wid = lax.axis_index("s") * NC + lax.axis_index("c")
        base = wid * b_per_w
        pltpu.sync_copy(idx_hbm.at[pl.ds(base, b_per_w)], idx_v)
        pltpu.async_copy(centers_hbm.at[idx_v], rows_v, sem).wait()
        pltpu.sync_copy(rows_v, out_hbm.at[pl.ds(base, b_per_w)])

    return gather_k


_BI = 512


def _tc_body(lcol_ref, lrow_ref, f_ref, cb_ref, res_ref, u_ref):
    B, D = f_ref.shape
    lrow = lrow_ref[...]
    d_all = cb_ref[...] - f_ref[...]
    for i in range(B // _BI):
        sl = pl.ds(i * _BI, _BI)
        lcol = lcol_ref[sl, :]
        eqf = (lcol == lrow).astype(jnp.float32)
        appear = jnp.sum(eqf, axis=1, keepdims=True)
        m = jax.lax.dot_general(
            eqf, d_all, (((1,), (0,)), ((), ())),
            preferred_element_type=jnp.float32)
        scale = _ALPHA / (1.0 + appear)
        cb_blk = cb_ref[sl, :]
        u_ref[sl, :] = cb_blk - scale * m
        r = f_ref[sl, :] - cb_blk
        res_ref[sl, :] = jnp.sum(r * r, axis=1, keepdims=True)


def _tc_math(labels, features, cb):
    B, D = features.shape
    lcol = labels.reshape(B, 1)
    lrow = labels.reshape(1, B)
    return pl.pallas_call(
        _tc_body,
        in_specs=[
            pl.BlockSpec((B, 1), lambda: (0, 0)),
            pl.BlockSpec((1, B), lambda: (0, 0)),
            pl.BlockSpec((B, D), lambda: (0, 0)),
            pl.BlockSpec((B, D), lambda: (0, 0)),
        ],
        out_specs=[
            pl.BlockSpec((B, 1), lambda: (0, 0)),
            pl.BlockSpec((B, D), lambda: (0, 0)),
        ],
        out_shape=[
            jax.ShapeDtypeStruct((B, 1), jnp.float32),
            jax.ShapeDtypeStruct((B, D), jnp.float32),
        ],
    )(lcol, lrow, features, cb)


_BR = 20000


def _copy_body(src_ref, dst_ref):
    dst_ref[...] = src_ref[...]


def _tc_copy(centers):
    C, D = centers.shape
    return pl.pallas_call(
        _copy_body,
        grid=(C // _BR,),
        in_specs=[pl.BlockSpec((_BR, D), lambda i: (i, 0))],
        out_specs=pl.BlockSpec((_BR, D), lambda i: (i, 0)),
        out_shape=jax.ShapeDtypeStruct((C, D), jnp.float32),
    )(centers)


def _make_scatter(C, D, B):
    NC, NS = 2, 16
    NW = NC * NS
    b_per_w = B // NW
    mesh = plsc.VectorSubcoreMesh(core_axis_name="c", subcore_axis_name="s")

    @functools.partial(
        pl.kernel,
        out_type=jax.ShapeDtypeStruct((b_per_w,), jnp.int32),
        mesh=mesh,
        scratch_types=[
            pltpu.VMEM((b_per_w,), jnp.int32),
            pltpu.VMEM((b_per_w, D), jnp.float32),
            pltpu.SemaphoreType.DMA,
        ],
        compiler_params=pltpu.CompilerParams(has_side_effects=True),
    )
    def scatter_k(table_hbm, idx_hbm, val_hbm, tok_hbm, idx_v, rows_v, sem):
        cid = lax.axis_index("c")
        sid = lax.axis_index("s")
        wid = sid * NC + cid
        base = wid * b_per_w
        pltpu.sync_copy(idx_hbm.at[pl.ds(base, b_per_w)], idx_v)
        pltpu.sync_copy(val_hbm.at[pl.ds(base, b_per_w)], rows_v)
        pltpu.async_copy(rows_v, table_hbm.at[idx_v], sem).wait()

        @pl.when(wid == 0)
        def _():
            pltpu.sync_copy(idx_v, tok_hbm)

    return scatter_k


def kernel(features, labels, centers):
    labels = labels.reshape(-1).astype(jnp.int32)
    features = features.astype(jnp.float32)
    B, D = features.shape
    C = centers.shape[0]

    cb = _make_gather(C, D, B)(centers, labels)
    result, u = _tc_math(labels, features, cb)
    table = _tc_copy(centers)
    tok = _make_scatter(C, D, B)(table, labels, u)
    table, _ = lax.optimization_barrier((table, tok))
    return (result, table)


# speedup vs baseline: 23.9379x; 1.1252x over previous
import functools

import jax
import jax.numpy as jnp
from jax import lax
from jax.experimental import pallas as pl
from jax.experimental.pallas import tpu as pltpu
from jax.experimental.pallas import tpu_sc as plsc

_ALPHA = 0.5


def _make_gather(C, D, B):
    NC, NS = 2, 16
    NW = NC * NS
    b_per_w = B // NW
    mesh = plsc.VectorSubcoreMesh(core_axis_name="c", subcore_axis_name="s")

    @functools.partial(
        pl.kernel,
        out_type=jax.ShapeDtypeStruct((B, D), jnp.float32),
        mesh=mesh,
        scratch_types=[
            pltpu.VMEM((b_per_w,), jnp.int32),
            pltpu.VMEM((b_per_w, D), jnp.float32),
            pltpu.SemaphoreType.DMA,
        ],
    )
    def gather_k(centers_hbm, idx_hbm, out_hbm, idx_v, rows_v, sem):
        wid = lax.axis_index("s") * NC + lax.axis_index("c")
        base = wid * b_per_w
        pltpu.sync_copy(idx_hbm.at[pl.ds(base, b_per_w)], idx_v)
        pltpu.async_copy(centers_hbm.at[idx_v], rows_v, sem).wait()
        pltpu.sync_copy(rows_v, out_hbm.at[pl.ds(base, b_per_w)])

    return gather_k


_BI = 512
_BR = 10000


def _fused_body(lcol_ref, lrow_ref, f_ref, cb_ref, src_ref,
                res_ref, u_ref, dst_ref):
    i = pl.program_id(0)
    nmb = f_ref.shape[0] // _BI
    dst_ref[...] = src_ref[...]

    @pl.when(i < nmb)
    def _():
        lrow = lrow_ref[...]
        sl = pl.ds(i * _BI, _BI)
        lcol = lcol_ref[sl, :]
        eqf = (lcol == lrow).astype(jnp.float32)
        appear = jnp.sum(eqf, axis=1, keepdims=True)
        d_all = cb_ref[...] - f_ref[...]
        m = jax.lax.dot_general(
            eqf, d_all, (((1,), (0,)), ((), ())),
            preferred_element_type=jnp.float32)
        scale = _ALPHA / (1.0 + appear)
        cb_blk = cb_ref[sl, :]
        u_ref[sl, :] = cb_blk - scale * m
        r = f_ref[sl, :] - cb_blk
        res_ref[sl, :] = jnp.sum(r * r, axis=1, keepdims=True)


def _tc_fused(labels, features, cb, centers):
    B, D = features.shape
    C = centers.shape[0]
    lcol = labels.reshape(B, 1)
    lrow = labels.reshape(1, B)
    return pl.pallas_call(
        _fused_body,
        grid=(C // _BR,),
        in_specs=[
            pl.BlockSpec((B, 1), lambda i: (0, 0)),
            pl.BlockSpec((1, B), lambda i: (0, 0)),
            pl.BlockSpec((B, D), lambda i: (0, 0)),
            pl.BlockSpec((B, D), lambda i: (0, 0)),
            pl.BlockSpec((_BR, D), lambda i: (i, 0)),
        ],
        out_specs=[
            pl.BlockSpec((B, 1), lambda i: (0, 0)),
            pl.BlockSpec((B, D), lambda i: (0, 0)),
            pl.BlockSpec((_BR, D), lambda i: (i, 0)),
        ],
        out_shape=[
            jax.ShapeDtypeStruct((B, 1), jnp.float32),
            jax.ShapeDtypeStruct((B, D), jnp.float32),
            jax.ShapeDtypeStruct((C, D), jnp.float32),
        ],
    )(lcol, lrow, features, cb, centers)


def _make_scatter(C, D, B):
    NC, NS = 2, 16
    NW = NC * NS
    b_per_w = B // NW
    mesh = plsc.VectorSubcoreMesh(core_axis_name="c", subcore_axis_name="s")

    @functools.partial(
        pl.kernel,
        out_type=jax.ShapeDtypeStruct((b_per_w,), jnp.int32),
        mesh=mesh,
        scratch_types=[
            pltpu.VMEM((b_per_w,), jnp.int32),
            pltpu.VMEM((b_per_w, D), jnp.float32),
            pltpu.SemaphoreType.DMA,
        ],
        compiler_params=pltpu.CompilerParams(has_side_effects=True),
    )
    def scatter_k(table_hbm, idx_hbm, val_hbm, tok_hbm, idx_v, rows_v, sem):
        cid = lax.axis_index("c")
        sid = lax.axis_index("s")
        wid = sid * NC + cid
        base = wid * b_per_w
        pltpu.sync_copy(idx_hbm.at[pl.ds(base, b_per_w)], idx_v)
        pltpu.sync_copy(val_hbm.at[pl.ds(base, b_per_w)], rows_v)
        pltpu.async_copy(rows_v, table_hbm.at[idx_v], sem).wait()

        @pl.when(wid == 0)
        def _():
            pltpu.sync_copy(idx_v, tok_hbm)

    return scatter_k


def kernel(features, labels, centers):
    labels = labels.reshape(-1).astype(jnp.int32)
    features = features.astype(jnp.float32)
    B, D = features.shape
    C = centers.shape[0]

    cb = _make_gather(C, D, B)(centers, labels)
    result, u, table = _tc_fused(labels, features, cb, centers)
    tok = _make_scatter(C, D, B)(table, labels, u)
    table, _ = lax.optimization_barrier((table, tok))
    return (result, table)


# speedup vs baseline: 24.5699x; 1.0264x over previous
import functools

import jax
import jax.numpy as jnp
from jax import lax
from jax.experimental import pallas as pl
from jax.experimental.pallas import tpu as pltpu
from jax.experimental.pallas import tpu_sc as plsc

_ALPHA = 0.5


def _make_gather(C, D, B):
    NC, NS = 2, 16
    NW = NC * NS
    b_per_w = B // NW
    mesh = plsc.VectorSubcoreMesh(core_axis_name="c", subcore_axis_name="s")

    @functools.partial(
        pl.kernel,
        out_type=jax.ShapeDtypeStruct((B, D), jnp.float32),
        mesh=mesh,
        scratch_types=[
            pltpu.VMEM((b_per_w,), jnp.int32),
            pltpu.VMEM((b_per_w, D), jnp.float32),
            pltpu.SemaphoreType.DMA,
        ],
    )
    def gather_k(centers_hbm, idx_hbm, out_hbm, idx_v, rows_v, sem):
        wid = lax.axis_index("s") * NC + lax.axis_index("c")
        base = wid * b_per_w
        pltpu.sync_copy(idx_hbm.at[pl.ds(base, b_per_w)], idx_v)
        pltpu.async_copy(centers_hbm.at[idx_v], rows_v, sem).wait()
        pltpu.sync_copy(rows_v, out_hbm.at[pl.ds(base, b_per_w)])

    return gather_k


_BI = 512
_BR = 10000
_NLEAD = 2


def _lead_body(src_ref, dst_ref):
    dst_ref[...] = src_ref[...]


def _tc_lead_copy(centers):
    C, D = centers.shape
    return pl.pallas_call(
        _lead_body,
        grid=(_NLEAD,),
        in_specs=[pl.BlockSpec((_BR, D), lambda i: (i, 0))],
        out_specs=pl.BlockSpec((_BR, D), lambda i: (i, 0)),
        out_shape=jax.ShapeDtypeStruct((C, D), jnp.float32),
    )(centers)


def _fused_body(lcol_ref, lrow_ref, f_ref, cb_ref, src_ref, tbl_in_ref,
                res_ref, u_ref, dst_ref):
    i = pl.program_id(0)
    nmb = f_ref.shape[0] // _BI
    dst_ref[...] = src_ref[...]

    @pl.when(i < nmb)
    def _():
        lrow = lrow_ref[...]
        sl = pl.ds(i * _BI, _BI)
        lcol = lcol_ref[sl, :]
        eqf = (lcol == lrow).astype(jnp.float32)
        appear = jnp.sum(eqf, axis=1, keepdims=True)
        d_all = cb_ref[...] - f_ref[...]
        m = jax.lax.dot_general(
            eqf, d_all, (((1,), (0,)), ((), ())),
            preferred_element_type=jnp.float32)
        scale = _ALPHA / (1.0 + appear)
        cb_blk = cb_ref[sl, :]
        u_ref[sl, :] = cb_blk - scale * m
        r = f_ref[sl, :] - cb_blk
        res_ref[sl, :] = jnp.sum(r * r, axis=1, keepdims=True)


def _tc_fused(labels, features, cb, centers, table0):
    B, D = features.shape
    C = centers.shape[0]
    lcol = labels.reshape(B, 1)
    lrow = labels.reshape(1, B)
    return pl.pallas_call(
        _fused_body,
        grid=(C // _BR - _NLEAD,),
        in_specs=[
            pl.BlockSpec((B, 1), lambda i: (0, 0)),
            pl.BlockSpec((1, B), lambda i: (0, 0)),
            pl.BlockSpec((B, D), lambda i: (0, 0)),
            pl.BlockSpec((B, D), lambda i: (0, 0)),
            pl.BlockSpec((_BR, D), lambda i: (i + _NLEAD, 0)),
            pl.BlockSpec(memory_space=pl.ANY),
        ],
        out_specs=[
            pl.BlockSpec((B, 1), lambda i: (0, 0)),
            pl.BlockSpec((B, D), lambda i: (0, 0)),
            pl.BlockSpec((_BR, D), lambda i: (i + _NLEAD, 0)),
        ],
        out_shape=[
            jax.ShapeDtypeStruct((B, 1), jnp.float32),
            jax.ShapeDtypeStruct((B, D), jnp.float32),
            jax.ShapeDtypeStruct((C, D), jnp.float32),
        ],
        input_output_aliases={5: 2},
    )(lcol, lrow, features, cb, centers, table0)


def _make_scatter(C, D, B):
    NC, NS = 2, 16
    NW = NC * NS
    b_per_w = B // NW
    mesh = plsc.VectorSubcoreMesh(core_axis_name="c", subcore_axis_name="s")

    @functools.partial(
        pl.kernel,
        out_type=jax.ShapeDtypeStruct((b_per_w,), jnp.int32),
        mesh=mesh,
        scratch_types=[
            pltpu.VMEM((b_per_w,), jnp.int32),
            pltpu.VMEM((b_per_w, D), jnp.float32),
            pltpu.SemaphoreType.DMA,
        ],
        compiler_params=pltpu.CompilerParams(has_side_effects=True),
    )
    def scatter_k(table_hbm, idx_hbm, val_hbm, tok_hbm, idx_v, rows_v, sem):
        cid = lax.axis_index("c")
        sid = lax.axis_index("s")
        wid = sid * NC + cid
        base = wid * b_per_w
        pltpu.sync_copy(idx_hbm.at[pl.ds(base, b_per_w)], idx_v)
        pltpu.sync_copy(val_hbm.at[pl.ds(base, b_per_w)], rows_v)
        pltpu.async_copy(rows_v, table_hbm.at[idx_v], sem).wait()

        @pl.when(wid == 0)
        def _():
            pltpu.sync_copy(idx_v, tok_hbm)

    return scatter_k


def kernel(features, labels, centers):
    labels = labels.reshape(-1).astype(jnp.int32)
    features = features.astype(jnp.float32)
    B, D = features.shape
    C = centers.shape[0]

    table0 = _tc_lead_copy(centers)
    cb = _make_gather(C, D, B)(centers, labels)
    result, u, table = _tc_fused(labels, features, cb, centers, table0)
    tok = _make_scatter(C, D, B)(table, labels, u)
    table, _ = lax.optimization_barrier((table, tok))
    return (result, table)
